# baseline (speedup 1.0000x reference)
import functools

import jax
import jax.numpy as jnp
from jax import lax
from jax.experimental import pallas as pl
from jax.experimental.pallas import tpu as pltpu
from jax.experimental.pallas import tpu_sc as plsc

BATCH = 4096
NUM_CLASSES = 1000
PAD_CLASSES = 1024
EMBED_DIM = 16

_SC_INFO = plsc.get_sparse_core_info()
_NC = _SC_INFO.num_cores
_NS = _SC_INFO.num_subcores
_NW = _NC * _NS
_BPW = BATCH // _NW


def _table_body(fc2_ref, prm_ref, out_ref):
    fcb = fc2_ref[...].astype(jnp.bfloat16)
    e0b = prm_ref[pl.ds(0, EMBED_DIM)].reshape(1, EMBED_DIM).astype(jnp.bfloat16)
    e1b = prm_ref[pl.ds(EMBED_DIM, EMBED_DIM)].reshape(1, EMBED_DIM).astype(jnp.bfloat16)
    dn = (((1,), (1,)), ((), ()))
    t0 = lax.dot_general(e0b, fcb, dn, preferred_element_type=jnp.float32)
    t1 = lax.dot_general(e1b, fcb, dn, preferred_element_type=jnp.float32)
    base = jnp.sum(t0) + prm_ref[2 * EMBED_DIM]
    out_ref[...] = (t1 - t0 + base).reshape(NUM_CLASSES)


def _lookup_kernel(table_hbm, idx_hbm, out_hbm, idx_v, out_v, sem):
    wid = lax.axis_index("s") * _NC + lax.axis_index("c")
    start = wid * _BPW
    pltpu.sync_copy(idx_hbm.at[pl.ds(start, _BPW)], idx_v)
    pltpu.async_copy(table_hbm.at[idx_v], out_v, sem).wait()
    pltpu.sync_copy(out_v, out_hbm.at[pl.ds(start, _BPW)])


@jax.jit
def kernel(sample_loc, embed_weight, fc_w, fc_b):
    params = jnp.concatenate(
        [
            embed_weight[0].astype(jnp.float32),
            embed_weight[1].astype(jnp.float32),
            fc_b.reshape(1).astype(jnp.float32),
        ]
    )
    table = pl.pallas_call(
        _table_body,
        out_shape=jax.ShapeDtypeStruct((NUM_CLASSES,), jnp.float32),
        in_specs=[
            pl.BlockSpec(memory_space=pltpu.VMEM),
            pl.BlockSpec(memory_space=pltpu.VMEM),
        ],
        out_specs=pl.BlockSpec(memory_space=pltpu.VMEM),
    )(fc_w.astype(jnp.float32).reshape(NUM_CLASSES, EMBED_DIM), params)

    idx = sample_loc.astype(jnp.int32)

    mesh = plsc.VectorSubcoreMesh(core_axis_name="c", subcore_axis_name="s")
    lookup = functools.partial(
        pl.kernel,
        mesh=mesh,
        out_type=jax.ShapeDtypeStruct((BATCH,), jnp.float32),
        scratch_types=[
            pltpu.VMEM((_BPW,), jnp.int32),
            pltpu.VMEM((_BPW,), jnp.float32),
            pltpu.SemaphoreType.DMA,
        ],
    )(_lookup_kernel)
    out = lookup(table, idx)
    return out.reshape(BATCH, 1)


# speedup vs baseline: 1.0773x; 1.0773x over previous
import functools

import jax
import jax.numpy as jnp
from jax import lax
from jax.experimental import pallas as pl
from jax.experimental.pallas import tpu as pltpu
from jax.experimental.pallas import tpu_sc as plsc

BATCH = 4096
NUM_CLASSES = 1000
PAD_CLASSES = 1024
EMBED_DIM = 16

_SC_INFO = plsc.get_sparse_core_info()
_NC = _SC_INFO.num_cores
_NS = _SC_INFO.num_subcores
_NW = 1 * _NS
_BPW = BATCH // _NW


def _table_body(fc2_ref, e01_ref, b_ref, out_ref):
    fcb = fc2_ref[...].astype(jnp.bfloat16)
    e0b = e01_ref[0:1, :].astype(jnp.bfloat16)
    e1b = e01_ref[1:2, :].astype(jnp.bfloat16)
    dn = (((1,), (1,)), ((), ()))
    t0 = lax.dot_general(e0b, fcb, dn, preferred_element_type=jnp.float32)
    t1 = lax.dot_general(e1b, fcb, dn, preferred_element_type=jnp.float32)
    base = jnp.sum(t0) + b_ref[0]
    out_ref[...] = (t1 - t0 + base).reshape(NUM_CLASSES)


def _lookup_kernel(table_hbm, idx_hbm, out_hbm, idx_v, out_v, sem):
    wid = lax.axis_index("s")
    start = wid * _BPW
    pltpu.sync_copy(idx_hbm.at[pl.ds(start, _BPW)], idx_v)
    pltpu.async_copy(table_hbm.at[idx_v], out_v, sem).wait()
    pltpu.sync_copy(out_v, out_hbm.at[pl.ds(start, _BPW)])


@jax.jit
def kernel(sample_loc, embed_weight, fc_w, fc_b):
    table = pl.pallas_call(
        _table_body,
        out_shape=jax.ShapeDtypeStruct((NUM_CLASSES,), jnp.float32),
        in_specs=[
            pl.BlockSpec(memory_space=pltpu.VMEM),
            pl.BlockSpec(memory_space=pltpu.VMEM),
            pl.BlockSpec(memory_space=pltpu.VMEM),
        ],
        out_specs=pl.BlockSpec(memory_space=pltpu.VMEM),
    )(fc_w.astype(jnp.float32).reshape(NUM_CLASSES, EMBED_DIM),
      embed_weight.astype(jnp.float32), fc_b.astype(jnp.float32))

    idx = sample_loc.astype(jnp.int32)

    mesh = plsc.VectorSubcoreMesh(core_axis_name="c", subcore_axis_name="s", num_cores=1)
    lookup = functools.partial(
        pl.kernel,
        mesh=mesh,
        out_type=jax.ShapeDtypeStruct((BATCH,), jnp.float32),
        scratch_types=[
            pltpu.VMEM((_BPW,), jnp.int32),
            pltpu.VMEM((_BPW,), jnp.float32),
            pltpu.SemaphoreType.DMA,
        ],
    )(_lookup_kernel)
    out = lookup(table, idx)
    return out.reshape(BATCH, 1)


# speedup vs baseline: 1.0777x; 1.0004x over previous
import functools

import jax
import jax.numpy as jnp
from jax import lax
from jax.experimental import pallas as pl
from jax.experimental.pallas import tpu as pltpu
from jax.experimental.pallas import tpu_sc as plsc

BATCH = 4096
NUM_CLASSES = 1000
PAD_CLASSES = 1024
EMBED_DIM = 16

_SC_INFO = plsc.get_sparse_core_info()
_NS = _SC_INFO.num_subcores
_NW = 1 * _NS
_BPW = BATCH // _NW


def _table_body(fc2_ref, e01_ref, b_ref, out_ref):
    fcb = fc2_ref[...].astype(jnp.bfloat16)
    e0b = e01_ref[0:1, :].astype(jnp.bfloat16)
    e1b = e01_ref[1:2, :].astype(jnp.bfloat16)
    dn = (((1,), (1,)), ((), ()))
    t0 = lax.dot_general(e0b, fcb, dn, preferred_element_type=jnp.float32)
    t1 = lax.dot_general(e1b, fcb, dn, preferred_element_type=jnp.float32)
    base = jnp.sum(t0) + b_ref[0]
    out_ref[...] = (t1 - t0 + base).reshape(NUM_CLASSES)


def _lookup_kernel(table_hbm, idx_hbm, out_hbm, idx_v, out_v, sem):
    wid = lax.axis_index("s")
    start = wid * _BPW
    pltpu.sync_copy(idx_hbm.at[pl.ds(start, _BPW)], idx_v)
    pltpu.async_copy(table_hbm.at[idx_v], out_v, sem).wait()
    pltpu.sync_copy(out_v, out_hbm.at[pl.ds(start, _BPW)])


@jax.jit
def kernel(sample_loc, embed_weight, fc_w, fc_b):
    table = pl.pallas_call(
        _table_body,
        out_shape=jax.ShapeDtypeStruct((NUM_CLASSES,), jnp.float32),
        in_specs=[
            pl.BlockSpec(memory_space=pltpu.VMEM),
            pl.BlockSpec(memory_space=pltpu.VMEM),
            pl.BlockSpec(memory_space=pltpu.VMEM),
        ],
        out_specs=pl.BlockSpec(memory_space=pltpu.VMEM),
    )(fc_w.astype(jnp.float32).reshape(NUM_CLASSES, EMBED_DIM),
      embed_weight.astype(jnp.float32), fc_b.astype(jnp.float32))

    idx = sample_loc.astype(jnp.int32)

    mesh = plsc.VectorSubcoreMesh(core_axis_name="c", subcore_axis_name="s", num_cores=1)
    lookup = functools.partial(
        pl.kernel,
        mesh=mesh,
        out_type=jax.ShapeDtypeStruct((BATCH,), jnp.float32),
        scratch_types=[
            pltpu.VMEM((_BPW,), jnp.int32),
            pltpu.VMEM((_BPW,), jnp.float32),
            pltpu.SemaphoreType.DMA,
        ],
    )(_lookup_kernel)
    out = lookup(table, idx)
    return out.reshape(BATCH, 1)
